# Initial kernel scaffold; baseline (speedup 1.0000x reference)
import jax
import jax.numpy as jnp
from jax.experimental import pallas as pl


def kernel(output, target, threshold):
    raise NotImplementedError("write your pallas kernel here")


# speedup vs baseline: 1.8898x; 1.8898x over previous
import functools

import jax
import jax.numpy as jnp
from jax import lax
from jax.experimental import pallas as pl
from jax.experimental.pallas import tpu as pltpu
from jax.experimental.pallas import tpu_sc as plsc

_L = 16
_NS = 16


def _row_stats_body(x_ref, tgt_ref, l_ref, m_full, m_excl, s_acc, t_acc,
                    *, V, W, nchunk):
    j = pl.program_id(0)

    @pl.when(j == 0)
    def _init():
        m_full[...] = jnp.full(m_full.shape, -jnp.inf, jnp.float32)
        m_excl[...] = jnp.full(m_excl.shape, -jnp.inf, jnp.float32)
        s_acc[...] = jnp.zeros(s_acc.shape, jnp.float32)
        t_acc[...] = jnp.zeros(t_acc.shape, jnp.float32)

    x = x_ref[...]
    rows = x.shape[0]
    col = j * W + lax.broadcasted_iota(jnp.int32, (rows, W), 1)
    valid = col < V
    eq = col == tgt_ref[...]
    ninf = jnp.float32(-jnp.inf)
    x_full = jnp.where(valid, x, ninf)
    x_excl = jnp.where(valid & jnp.logical_not(eq), x, ninf)
    m_old = m_full[...]
    m_new = jnp.maximum(m_old, jnp.max(x_full, axis=1, keepdims=True))
    s_acc[...] = (s_acc[...] * jnp.exp(m_old - m_new)
                  + jnp.sum(jnp.exp(x_full - m_new), axis=1, keepdims=True))
    m_full[...] = m_new
    m_excl[...] = jnp.maximum(m_excl[...],
                              jnp.max(x_excl, axis=1, keepdims=True))
    t_acc[...] = t_acc[...] + jnp.sum(jnp.where(eq, x_full, 0.0), axis=1,
                                      keepdims=True)

    @pl.when(j == nchunk - 1)
    def _finish():
        t = t_acc[...]
        lse = m_full[...] + jnp.log(s_acc[...])
        margin = t - m_excl[...]
        fst = jnp.maximum(1.0 - margin, 0.0)
        snd = jnp.maximum(1.0 - t + lse, 0.0)
        l_ref[...] = jnp.where(margin >= 0.0, fst, snd)


def _row_losses(output, tgt2d, W=2048):
    B, V = output.shape
    nchunk = pl.cdiv(V, W)
    body = functools.partial(_row_stats_body, V=V, W=W, nchunk=nchunk)
    return pl.pallas_call(
        body,
        grid=(nchunk,),
        in_specs=[
            pl.BlockSpec((B, W), lambda j: (0, j)),
            pl.BlockSpec((B, 1), lambda j: (0, 0)),
        ],
        out_specs=pl.BlockSpec((B, 1), lambda j: (0, 0)),
        out_shape=jax.ShapeDtypeStruct((B, 1), jnp.float32),
        scratch_shapes=[pltpu.VMEM((B, 1), jnp.float32) for _ in range(4)],
    )(output, tgt2d)


_GDN = lax.GatherDimensionNumbers(offset_dims=(), collapsed_slice_dims=(0,),
                                  start_index_map=(0,))


def _lane_shuffle(x, idx):
    return lax.gather(x, idx[:, None], _GDN, (1,),
                      mode=lax.GatherScatterMode.PROMISE_IN_BOUNDS)


def _lane_sum(x, lanes):
    for sh in (8, 4, 2, 1):
        x = x + _lane_shuffle(x, jnp.bitwise_xor(lanes, sh))
    return x


def _make_sc_select(B):
    mesh = plsc.VectorSubcoreMesh(core_axis_name="c", subcore_axis_name="s")
    ipw = B // _NS
    nj = B // _L

    @functools.partial(
        pl.kernel,
        mesh=mesh,
        out_type=jax.ShapeDtypeStruct((_L,), jnp.float32),
        scratch_types=[
            pltpu.VMEM((B,), jnp.float32),
            pltpu.VMEM((_L,), jnp.float32),
            pltpu.VMEM((2, _L), jnp.float32),
            pltpu.VMEM((_NS, 2, _L), jnp.float32),
            pltpu.VMEM((_L,), jnp.float32),
            pltpu.VMEM_SHARED((_NS, 2, _L), jnp.float32),
        ],
    )
    def sc_select(l_hbm, thr_hbm, out_hbm, l_v, thr_v, part_v, all_v, out_v,
                  shared):
        cid = lax.axis_index("c")
        sid = lax.axis_index("s")
        pltpu.sync_copy(l_hbm, l_v)
        pltpu.sync_copy(thr_hbm, thr_v)
        lanes = lax.iota(jnp.int32, _L)
        thr_vec = thr_v[...]
        base = sid * ipw
        zero = jnp.zeros((_L,), jnp.float32)
        one = jnp.full((_L,), 1.0, jnp.float32)

        def ubody(u, carry):
            off_u = pl.multiple_of(base + u * _L, _L)
            mi = l_v[pl.ds(off_u, _L)]

            def rbody(r, carry2):
                c1, kc = carry2
                li = _lane_shuffle(mi, jnp.full((_L,), r, jnp.int32))
                igv = jnp.full((_L,), base + u * _L + r, jnp.int32)

                def jbody(jv, jcarry):
                    s_par, r_par = jcarry
                    off = pl.multiple_of(jv * _L, _L)
                    lj = l_v[pl.ds(off, _L)]
                    jidx = jnp.full((_L,), jv * _L, jnp.int32) + lanes
                    lt = lj < li
                    tie = jnp.logical_and(lj == li, jidx < igv)
                    take = jnp.logical_or(lt, tie)
                    s_par = s_par + jnp.where(take, lj, zero)
                    r_par = r_par + jnp.where(take, one, zero)
                    return s_par, r_par

                s_par, r_par = lax.fori_loop(0, nj, jbody, (zero, zero))
                s_i = _lane_sum(s_par, lanes)
                r_i = _lane_sum(r_par, lanes)
                kept = (s_i + li) <= (thr_vec - r_i)
                c1 = c1 + jnp.where(kept, li, zero)
                kc = kc + jnp.where(kept, one, zero)
                return c1, kc

            return lax.fori_loop(0, _L, rbody, carry)

        c1, kc = lax.fori_loop(0, ipw // _L, ubody, (zero, zero))
        part_v[0] = c1
        part_v[1] = kc
        pltpu.sync_copy(part_v, shared.at[sid])
        plsc.subcore_barrier()

        @pl.when(jnp.logical_and(cid == 0, sid == 0))
        def _reduce():
            pltpu.sync_copy(shared, all_v)
            c1v = jnp.zeros((_L,), jnp.float32)
            kv = jnp.zeros((_L,), jnp.float32)
            for w in range(_NS):
                c1v = c1v + all_v[w, 0]
                kv = kv + all_v[w, 1]
            c2v = jnp.float32(B) - kv
            out_v[...] = jnp.where(c1v < c2v, c2v, c1v)
            pltpu.sync_copy(out_v, out_hbm)

    return sc_select


def kernel(output, target, threshold):
    B, V = output.shape
    tgt2d = target.astype(jnp.int32).reshape(B, 1)
    losses = _row_losses(output, tgt2d)
    thr_vec = jnp.full((_L,), threshold, dtype=jnp.float32)
    out16 = _make_sc_select(B)(losses.reshape(B), thr_vec)
    return out16[0]
